# baseline (speedup 1.0000x reference)
import jax
import jax.numpy as jnp
from jax.experimental import pallas as pl
from jax.experimental.pallas import tpu as pltpu

_B, _N = 32, 4096
_G = 256
_K = 32


def _fps_kernel(xs_ref, ys_ref, zs_ref, fidx_ref, cx_ref, cy_ref, cz_ref):
    xs = xs_ref[...]
    ys = ys_ref[...]
    zs = zs_ref[...]
    iota = jax.lax.broadcasted_iota(jnp.int32, (_B, _N), 1)
    iota_g = jax.lax.broadcasted_iota(jnp.int32, (_B, _G), 1)

    fidx_ref[...] = jnp.zeros((_B, _G), jnp.int32)
    cx_ref[...] = jnp.zeros((_B, _G), jnp.float32)
    cy_ref[...] = jnp.zeros((_B, _G), jnp.float32)
    cz_ref[...] = jnp.zeros((_B, _G), jnp.float32)

    def body(i, state):
        dists, farthest = state
        mask = iota == farthest
        cx = jnp.sum(jnp.where(mask, xs, 0.0), axis=1, keepdims=True)
        cy = jnp.sum(jnp.where(mask, ys, 0.0), axis=1, keepdims=True)
        cz = jnp.sum(jnp.where(mask, zs, 0.0), axis=1, keepdims=True)
        dx = xs - cx
        dy = ys - cy
        dz = zs - cz
        d = dx * dx + dy * dy + dz * dz
        dists = jnp.minimum(dists, d)
        slot_i = (iota_g == i).astype(jnp.int32)
        slot_f = slot_i.astype(jnp.float32)
        fidx_ref[...] = fidx_ref[...] + slot_i * farthest
        cx_ref[...] = cx_ref[...] + slot_f * cx
        cy_ref[...] = cy_ref[...] + slot_f * cy
        cz_ref[...] = cz_ref[...] + slot_f * cz
        m = jnp.max(dists, axis=1, keepdims=True)
        new_far = jnp.min(
            jnp.where(dists == m, iota, _N), axis=1, keepdims=True
        )
        return dists, new_far

    dists0 = jnp.full((_B, _N), 1e10, dtype=jnp.float32)
    far0 = jnp.zeros((_B, 1), jnp.int32)
    jax.lax.fori_loop(0, _G, body, (dists0, far0), unroll=16)


def _knn_kernel(cx_ref, cy_ref, cz_ref, xs_ref, ys_ref, zs_ref, idx_ref):
    cx = cx_ref[0]
    cy = cy_ref[0]
    cz = cz_ref[0]
    xs = xs_ref[0]
    ys = ys_ref[0]
    zs = zs_ref[0]
    qn = cx * cx + cy * cy + cz * cz
    rn = xs * xs + ys * ys + zs * zs
    bf = lambda v: v.astype(jnp.bfloat16).astype(jnp.float32)
    qr = bf(cx) * bf(xs) + bf(cy) * bf(ys) + bf(cz) * bf(zs)
    d = (qn + rn) - 2.0 * qr

    iota_n = jax.lax.broadcasted_iota(jnp.int32, (_G, _N), 1)
    iota_k = jax.lax.broadcasted_iota(jnp.int32, (_G, _K), 1)
    idx_ref[0] = jnp.zeros((_G, _K), jnp.int32)

    def body(k, d):
        m = jnp.min(d, axis=1, keepdims=True)
        pos = jnp.min(jnp.where(d == m, iota_n, _N), axis=1, keepdims=True)
        slot = (iota_k == k).astype(jnp.int32)
        idx_ref[0] = idx_ref[0] + slot * pos
        return jnp.where(iota_n == pos, jnp.inf, d)

    jax.lax.fori_loop(0, _K, body, d, unroll=32)


def kernel(xyz):
    xt = jnp.transpose(xyz, (2, 0, 1))
    xs, ys, zs = xt[0], xt[1], xt[2]
    fidx, cx, cy, cz = pl.pallas_call(
        _fps_kernel,
        out_shape=[
            jax.ShapeDtypeStruct((_B, _G), jnp.int32),
            jax.ShapeDtypeStruct((_B, _G), jnp.float32),
            jax.ShapeDtypeStruct((_B, _G), jnp.float32),
            jax.ShapeDtypeStruct((_B, _G), jnp.float32),
        ],
    )(xs, ys, zs)
    del fidx
    center = jnp.stack([cx, cy, cz], axis=-1)

    c3 = (cx[:, :, None], cy[:, :, None], cz[:, :, None])
    idx = pl.pallas_call(
        _knn_kernel,
        grid=(_B,),
        in_specs=[
            pl.BlockSpec((1, _G, 1), lambda b: (b, 0, 0)),
            pl.BlockSpec((1, _G, 1), lambda b: (b, 0, 0)),
            pl.BlockSpec((1, _G, 1), lambda b: (b, 0, 0)),
            pl.BlockSpec((1, 1, _N), lambda b: (b, 0, 0)),
            pl.BlockSpec((1, 1, _N), lambda b: (b, 0, 0)),
            pl.BlockSpec((1, 1, _N), lambda b: (b, 0, 0)),
        ],
        out_specs=pl.BlockSpec((1, _G, _K), lambda b: (b, 0, 0)),
        out_shape=jax.ShapeDtypeStruct((_B, _G, _K), jnp.int32),
        compiler_params=pltpu.CompilerParams(
            dimension_semantics=("parallel",),
        ),
    )(*c3, xs[:, None, :], ys[:, None, :], zs[:, None, :])
    return (idx, center)


# speedup vs baseline: 1.2112x; 1.2112x over previous
import jax
import jax.numpy as jnp
from jax.experimental import pallas as pl
from jax.experimental.pallas import tpu as pltpu

_B, _N = 32, 4096
_G = 256
_K = 32


def _fps_kernel(xs_ref, ys_ref, zs_ref, fidx_ref, cx_ref, cy_ref, cz_ref):
    xs = xs_ref[...]
    ys = ys_ref[...]
    zs = zs_ref[...]
    iota = jax.lax.broadcasted_iota(jnp.int32, (_B, _N), 1)
    iota_g = jax.lax.broadcasted_iota(jnp.int32, (_B, _G), 1)

    fidx_ref[...] = jnp.zeros((_B, _G), jnp.int32)
    cx_ref[...] = jnp.zeros((_B, _G), jnp.float32)
    cy_ref[...] = jnp.zeros((_B, _G), jnp.float32)
    cz_ref[...] = jnp.zeros((_B, _G), jnp.float32)

    def body(i, state):
        dists, farthest = state
        mask = iota == farthest
        cx = jnp.sum(jnp.where(mask, xs, 0.0), axis=1, keepdims=True)
        cy = jnp.sum(jnp.where(mask, ys, 0.0), axis=1, keepdims=True)
        cz = jnp.sum(jnp.where(mask, zs, 0.0), axis=1, keepdims=True)
        dx = xs - cx
        dy = ys - cy
        dz = zs - cz
        d = dx * dx + dy * dy + dz * dz
        dists = jnp.minimum(dists, d)
        slot_i = (iota_g == i).astype(jnp.int32)
        slot_f = slot_i.astype(jnp.float32)
        fidx_ref[...] = fidx_ref[...] + slot_i * farthest
        cx_ref[...] = cx_ref[...] + slot_f * cx
        cy_ref[...] = cy_ref[...] + slot_f * cy
        cz_ref[...] = cz_ref[...] + slot_f * cz
        m = jnp.max(dists, axis=1, keepdims=True)
        new_far = jnp.min(
            jnp.where(dists == m, iota, _N), axis=1, keepdims=True
        )
        return dists, new_far

    dists0 = jnp.full((_B, _N), 1e10, dtype=jnp.float32)
    far0 = jnp.zeros((_B, 1), jnp.int32)
    jax.lax.fori_loop(0, _G, body, (dists0, far0), unroll=8)


def _knn_kernel(cx_ref, cy_ref, cz_ref, xs_ref, ys_ref, zs_ref, idx_ref):
    cx = cx_ref[0]
    cy = cy_ref[0]
    cz = cz_ref[0]
    xs = xs_ref[0]
    ys = ys_ref[0]
    zs = zs_ref[0]
    qn = cx * cx + cy * cy + cz * cz
    rn = xs * xs + ys * ys + zs * zs
    bf = lambda v: v.astype(jnp.bfloat16).astype(jnp.float32)
    qr = bf(cx) * bf(xs) + bf(cy) * bf(ys) + bf(cz) * bf(zs)
    d = (qn + rn) - 2.0 * qr

    iota_n = jax.lax.broadcasted_iota(jnp.int32, (_G, _N), 1)
    iota_k = jax.lax.broadcasted_iota(jnp.int32, (_G, _K), 1)
    idx_ref[0] = jnp.zeros((_G, _K), jnp.int32)

    def body(k, d):
        m = jnp.min(d, axis=1, keepdims=True)
        pos = jnp.min(jnp.where(d == m, iota_n, _N), axis=1, keepdims=True)
        slot = (iota_k == k).astype(jnp.int32)
        idx_ref[0] = idx_ref[0] + slot * pos
        return jnp.where(iota_n == pos, jnp.inf, d)

    jax.lax.fori_loop(0, _K, body, d, unroll=16)


def kernel(xyz):
    xt = jnp.transpose(xyz, (2, 0, 1))
    xs, ys, zs = xt[0], xt[1], xt[2]
    fidx, cx, cy, cz = pl.pallas_call(
        _fps_kernel,
        out_shape=[
            jax.ShapeDtypeStruct((_B, _G), jnp.int32),
            jax.ShapeDtypeStruct((_B, _G), jnp.float32),
            jax.ShapeDtypeStruct((_B, _G), jnp.float32),
            jax.ShapeDtypeStruct((_B, _G), jnp.float32),
        ],
    )(xs, ys, zs)
    del fidx
    center = jnp.stack([cx, cy, cz], axis=-1)

    c3 = (cx[:, :, None], cy[:, :, None], cz[:, :, None])
    idx = pl.pallas_call(
        _knn_kernel,
        grid=(_B,),
        in_specs=[
            pl.BlockSpec((1, _G, 1), lambda b: (b, 0, 0)),
            pl.BlockSpec((1, _G, 1), lambda b: (b, 0, 0)),
            pl.BlockSpec((1, _G, 1), lambda b: (b, 0, 0)),
            pl.BlockSpec((1, 1, _N), lambda b: (b, 0, 0)),
            pl.BlockSpec((1, 1, _N), lambda b: (b, 0, 0)),
            pl.BlockSpec((1, 1, _N), lambda b: (b, 0, 0)),
        ],
        out_specs=pl.BlockSpec((1, _G, _K), lambda b: (b, 0, 0)),
        out_shape=jax.ShapeDtypeStruct((_B, _G, _K), jnp.int32),
        compiler_params=pltpu.CompilerParams(
            dimension_semantics=("parallel",),
        ),
    )(*c3, xs[:, None, :], ys[:, None, :], zs[:, None, :])
    return (idx, center)
